# baseline (speedup 1.0000x reference)
import functools

import jax
import jax.numpy as jnp
from jax import lax
from jax.experimental import pallas as pl
from jax.experimental.pallas import tpu as pltpu
from jax.experimental.pallas import tpu_sc as plsc

N = 10000
E = 320000
D = 128
NC = 2
NS = 16
NW = NC * NS
NP = 10240
RPT_DEG = NP // NS
EPT2 = E // NS
K = 80
EPT = E // NW
CH = EPT // K
R = 3840
NPASS = 3
ACC = 3920
DUMP = R

f32 = jnp.float32

_mesh = plsc.VectorSubcoreMesh(
    core_axis_name="c", subcore_axis_name="s", num_cores=NC, num_subcores=NS)
_sc_params = pltpu.CompilerParams(needs_layout_passes=False)


@functools.partial(
    pl.kernel,
    out_type=[
        jax.ShapeDtypeStruct((NW, NP), f32),
        jax.ShapeDtypeStruct((2, NP), f32),
    ],
    mesh=_mesh,
    scratch_types=[
        pltpu.VMEM((EPT2,), jnp.int32),
        pltpu.VMEM((NP,), f32),
        pltpu.VMEM((RPT_DEG,), f32),
        pltpu.VMEM((RPT_DEG,), f32),
    ],
    compiler_params=_sc_params,
)
def _deg_kernel(idx_hbm, part_out, deg_out, idx_v, deg_v, acc_v, tmp_v):
    c = lax.axis_index("c")
    s = lax.axis_index("s")
    row = c * NS + s
    pltpu.sync_copy(idx_hbm.at[row], idx_v)

    zero16 = jnp.zeros((16,), f32)
    ones16 = jnp.ones((16,), f32)

    def zbody(i, carry):
        deg_v[pl.ds(i * 16, 16)] = zero16
        return carry
    lax.fori_loop(0, NP // 16, zbody, None)

    def ebody(e, carry):
        idx = idx_v[pl.ds(e * 16, 16)]
        plsc.addupdate_scatter(deg_v, [idx], ones16)
        return carry
    lax.fori_loop(0, EPT2 // 16, ebody, None)

    pltpu.sync_copy(deg_v, part_out.at[row])
    plsc.subcore_barrier()

    base = s * RPT_DEG
    pltpu.sync_copy(part_out.at[c * NS, pl.ds(base, RPT_DEG)], acc_v)
    for p in range(1, NS):
        pltpu.sync_copy(part_out.at[c * NS + p, pl.ds(base, RPT_DEG)], tmp_v)

        def abody(i, carry):
            sl = pl.ds(i * 16, 16)
            acc_v[sl] = acc_v[sl] + tmp_v[sl]
            return carry
        lax.fori_loop(0, RPT_DEG // 16, abody, None)
    pltpu.sync_copy(acc_v, deg_out.at[c, pl.ds(base, RPT_DEG)])


KS = 128
LSZ = 10624
DPAD = 10016


@functools.partial(
    pl.kernel,
    out_type=jax.ShapeDtypeStruct((NC, N, D), f32),
    mesh=_mesh,
    scratch_types=[
        pltpu.VMEM((LSZ,), jnp.int32),
        pltpu.VMEM((LSZ,), jnp.int32),
        pltpu.VMEM((LSZ,), jnp.int32),
        pltpu.VMEM((LSZ,), jnp.int32),
        pltpu.VMEM((LSZ,), jnp.int32),
        pltpu.VMEM((KS, D), f32),
        pltpu.VMEM((KS, D), f32),
        pltpu.VMEM((K, D), f32),
        pltpu.VMEM_SHARED((ACC, D), f32),
        pltpu.SemaphoreType.DMA,
        pltpu.SemaphoreType.DMA,
    ],
    compiler_params=_sc_params,
)
def _scatter_kernel(src_hbm, dst_hbm, h_hbm, out_hbm,
                    src_v, dst_v, dl0, dl1, dl2,
                    rows_a, rows_b, zbuf, acc_sh, sem_a, sem_b):
    c = lax.axis_index("c")
    s = lax.axis_index("s")
    w = c * NS + s
    pltpu.sync_copy(src_hbm.at[w], src_v)
    pltpu.sync_copy(dst_hbm.at[w], dst_v)

    dumpv = jnp.full((16,), DUMP, jnp.int32)
    r1 = jnp.full((16,), R, jnp.int32)
    r2 = jnp.full((16,), 2 * R, jnp.int32)

    def tbody(i, carry):
        sl = pl.ds(i * 16, 16)
        d = dst_v[sl]
        dl0[sl] = jnp.where(d < r1, d, dumpv)
        in1 = (d >= r1) & (d < r2)
        dl1[sl] = jnp.where(in1, d - r1, dumpv)
        d2 = d - r2
        dl2[sl] = jnp.where(d >= r2, d2, dumpv)
        return carry
    lax.fori_loop(0, LSZ // 16, tbody, None)

    zero16 = jnp.zeros((16,), f32)

    def zrow(i, carry):
        for j in range(D // 16):
            zbuf[i, pl.ds(j * 16, 16)] = zero16
        return carry
    lax.fori_loop(0, K, zrow, None)

    def zero_acc():
        for i in range(-(-(ACC // K) // NS)):
            m = i * NS + s

            @pl.when(m < ACC // K)
            def _():
                pltpu.sync_copy(zbuf, acc_sh.at[pl.ds(m * K, K)])

    zero_acc()
    plsc.subcore_barrier()

    NCH = 10240 // KS
    PAIRS = NCH // 2

    for p, dlist in enumerate((dl0, dl1, dl2)):
        pltpu.async_copy(h_hbm.at[src_v.at[pl.ds(0, KS)]], rows_a, sem_a)

        def pair(j2, carry, dlist=dlist):
            j = j2 * 2 * KS
            pltpu.make_async_copy(
                h_hbm.at[src_v.at[pl.ds(j, KS)]], rows_a, sem_a).wait()
            pltpu.async_copy(
                h_hbm.at[src_v.at[pl.ds(j + KS, KS)]], rows_b, sem_b)
            pltpu.sync_copy(rows_a, acc_sh.at[dlist.at[pl.ds(j, KS)]],
                            add=True)
            pltpu.make_async_copy(
                h_hbm.at[src_v.at[pl.ds(j + KS, KS)]], rows_b, sem_b).wait()
            pltpu.async_copy(
                h_hbm.at[src_v.at[pl.ds(j + 2 * KS, KS)]], rows_a, sem_a)
            pltpu.sync_copy(rows_b, acc_sh.at[dlist.at[pl.ds(j + KS, KS)]],
                            add=True)
            return carry
        lax.fori_loop(0, PAIRS, pair, None)
        pltpu.make_async_copy(
            h_hbm.at[src_v.at[pl.ds(0, KS)]], rows_a, sem_a).wait()

        plsc.subcore_barrier()

        rp = min(R, N - p * R)
        cp = rp // K
        for i in range(-(-cp // NS)):
            m = i * NS + s

            @pl.when(m < cp)
            def _(m=m):
                pltpu.sync_copy(acc_sh.at[pl.ds(m * K, K)], zbuf)
                pltpu.sync_copy(zbuf, out_hbm.at[c, pl.ds(p * R + m * K, K)])

        if p < NPASS - 1:
            lax.fori_loop(0, K, zrow, None)
            zero_acc()
            plsc.subcore_barrier()


_BLK = 2000
_GRID = N // _BLK


def _mm_scale_body(x_ref, w_ref, b_ref, degs_ref, o_ref):
    h = jnp.dot(x_ref[...], w_ref[...], preferred_element_type=f32) + b_ref[...]
    o_ref[...] = h * lax.rsqrt(jnp.maximum(degs_ref[...], 1.0))


def _tc_mm_scale(x, w, b2d, degs):
    return pl.pallas_call(
        _mm_scale_body,
        grid=(_GRID,),
        in_specs=[
            pl.BlockSpec((_BLK, D), lambda i: (i, 0)),
            pl.BlockSpec((D, D), lambda i: (0, 0)),
            pl.BlockSpec((1, D), lambda i: (0, 0)),
            pl.BlockSpec((_BLK, 1), lambda i: (i, 0)),
        ],
        out_specs=pl.BlockSpec((_BLK, D), lambda i: (i, 0)),
        out_shape=jax.ShapeDtypeStruct((N, D), f32),
    )(x, w, b2d, degs)


def _post_body(p_ref, degd_ref, o_ref):
    a = (p_ref[0] + p_ref[1]) * lax.rsqrt(jnp.maximum(degd_ref[...], 1.0))
    o_ref[...] = jnp.where(a >= 0, a, 0.01 * a)


def _tc_post(p, degd):
    return pl.pallas_call(
        _post_body,
        grid=(_GRID,),
        in_specs=[
            pl.BlockSpec((NC, _BLK, D), lambda i: (0, i, 0)),
            pl.BlockSpec((_BLK, 1), lambda i: (i, 0)),
        ],
        out_specs=pl.BlockSpec((_BLK, D), lambda i: (i, 0)),
        out_shape=jax.ShapeDtypeStruct((N, D), f32),
    )(p, degd)


def _fin_body(x0_ref, ys_ref, o_ref):
    o_ref[...] = (x0_ref[...] + ys_ref[0] + ys_ref[1]) * (1.0 / 3.0)


def _tc_fin(x0, ys):
    return pl.pallas_call(
        _fin_body,
        grid=(_GRID,),
        in_specs=[
            pl.BlockSpec((_BLK, D), lambda i: (i, 0)),
            pl.BlockSpec((2, _BLK, D), lambda i: (0, i, 0)),
        ],
        out_specs=pl.BlockSpec((_BLK, D), lambda i: (i, 0)),
        out_shape=jax.ShapeDtypeStruct((N, D), f32),
    )(x0, ys)


def kernel(edge_index, all_embed, W1, b1, W2, b2):
    ei = edge_index.astype(jnp.int32)
    deg_idx = ei.reshape(NW, EPT2)
    pad_s = jnp.zeros((NW, LSZ - EPT), jnp.int32)
    pad_d = jnp.full((NW, LSZ - EPT), DPAD, jnp.int32)
    src_r = jnp.concatenate((ei[0].reshape(NW, EPT), pad_s), axis=1)
    dst_r = jnp.concatenate((ei[1].reshape(NW, EPT), pad_d), axis=1)

    _, degs = _deg_kernel(deg_idx)
    deg_src = degs[0, :N].reshape(N, 1)
    deg_dst = degs[1, :N].reshape(N, 1)
    Ws = jnp.stack((W1, W2))
    bs = jnp.stack((b1.reshape(1, D), b2.reshape(1, D)))

    def layer(x, wb):
        w, b2d = wb
        h = _tc_mm_scale(x, w, b2d, deg_src)
        p = _scatter_kernel(src_r, dst_r, h)
        xn = _tc_post(p, deg_dst)
        return xn, xn

    _, ys = lax.scan(layer, all_embed, (Ws, bs))
    return _tc_fin(all_embed, ys)


# speedup vs baseline: 4.8224x; 4.8224x over previous
import functools

import jax
import jax.numpy as jnp
from jax import lax
from jax.experimental import pallas as pl
from jax.experimental.pallas import tpu as pltpu
from jax.experimental.pallas import tpu_sc as plsc

N = 10000
E = 320000
D = 128
NC = 2
NS = 16
NW = NC * NS
NP = 10240
RPT_DEG = NP // NS
EPT2 = E // NS
K = 80
EPT = E // NW
CH = EPT // K
R = 3840
NPASS = 3
ACC = 3920
DUMP = R

f32 = jnp.float32

_mesh = plsc.VectorSubcoreMesh(
    core_axis_name="c", subcore_axis_name="s", num_cores=NC, num_subcores=NS)
_sc_params = pltpu.CompilerParams(needs_layout_passes=False)


@functools.partial(
    pl.kernel,
    out_type=[
        jax.ShapeDtypeStruct((NW, NP), f32),
        jax.ShapeDtypeStruct((2, NP), f32),
    ],
    mesh=_mesh,
    scratch_types=[
        pltpu.VMEM((EPT2,), jnp.int32),
        pltpu.VMEM((NP,), f32),
        pltpu.VMEM((RPT_DEG,), f32),
        pltpu.VMEM((RPT_DEG,), f32),
    ],
    compiler_params=_sc_params,
)
def _deg_kernel(idx_hbm, part_out, deg_out, idx_v, deg_v, acc_v, tmp_v):
    c = lax.axis_index("c")
    s = lax.axis_index("s")
    row = c * NS + s
    pltpu.sync_copy(idx_hbm.at[row], idx_v)

    zero16 = jnp.zeros((16,), f32)
    ones16 = jnp.ones((16,), f32)

    def zbody(i, carry):
        deg_v[pl.ds(i * 16, 16)] = zero16
        return carry
    lax.fori_loop(0, NP // 16, zbody, None)

    def ebody(e, carry):
        idx = idx_v[pl.ds(e * 16, 16)]
        plsc.addupdate_scatter(deg_v, [idx], ones16)
        return carry
    lax.fori_loop(0, EPT2 // 16, ebody, None)

    pltpu.sync_copy(deg_v, part_out.at[row])
    plsc.subcore_barrier()

    base = s * RPT_DEG
    pltpu.sync_copy(part_out.at[c * NS, pl.ds(base, RPT_DEG)], acc_v)
    for p in range(1, NS):
        pltpu.sync_copy(part_out.at[c * NS + p, pl.ds(base, RPT_DEG)], tmp_v)

        def abody(i, carry):
            sl = pl.ds(i * 16, 16)
            acc_v[sl] = acc_v[sl] + tmp_v[sl]
            return carry
        lax.fori_loop(0, RPT_DEG // 16, abody, None)
    pltpu.sync_copy(acc_v, deg_out.at[c, pl.ds(base, RPT_DEG)])


@functools.partial(
    pl.kernel,
    out_type=jax.ShapeDtypeStruct((NC, N, D), f32),
    mesh=_mesh,
    scratch_types=[
        pltpu.VMEM((CH, K), jnp.int32),
        pltpu.VMEM((CH, K), jnp.int32),
        pltpu.VMEM((CH, K), jnp.int32),
        pltpu.VMEM((CH, K), jnp.int32),
        pltpu.VMEM((K, D), f32),
        pltpu.VMEM((K, D), f32),
        pltpu.VMEM((K, D), f32),
        pltpu.VMEM_SHARED((ACC, D), f32),
        pltpu.SemaphoreType.DMA,
        pltpu.SemaphoreType.DMA,
    ],
    compiler_params=_sc_params,
)
def _scatter_kernel(src_hbm, dst_hbm, h_hbm, out_hbm,
                    src_v, dst0_v, dst1_v, dst2_v, rows_a, rows_b, zbuf,
                    acc_sh, sem_a, sem_b):
    c = lax.axis_index("c")
    s = lax.axis_index("s")
    w = c * NS + s
    pltpu.sync_copy(src_hbm.at[w], src_v)
    pltpu.sync_copy(dst_hbm.at[w], dst0_v)

    dumpv = jnp.full((16,), DUMP, jnp.int32)
    m63 = jnp.full((16,), 63, jnp.int32)
    r1 = jnp.full((16,), R, jnp.int32)
    r2 = jnp.full((16,), 2 * R, jnp.int32)

    def tbody(j, carry):
        for k in range(K // 16):
            sl = pl.ds(k * 16, 16)
            d = dst0_v[j, sl]
            dmp = dumpv + (d & m63)
            dst2_v[j, sl] = jnp.where(d >= r2, d - r2, dmp)
            in1 = (d >= r1) & (d < r2)
            dst1_v[j, sl] = jnp.where(in1, d - r1, dmp)
            dst0_v[j, sl] = jnp.where(d < r1, d, dmp)
        return carry
    lax.fori_loop(0, CH, tbody, None)

    zero16 = jnp.zeros((16,), f32)

    def zrow(i, carry):
        for j in range(D // 16):
            zbuf[i, pl.ds(j * 16, 16)] = zero16
        return carry
    lax.fori_loop(0, K, zrow, None)

    def zero_acc():
        for i in range(-(-(ACC // K) // NS)):
            m = i * NS + s

            @pl.when(m < ACC // K)
            def _():
                pltpu.sync_copy(zbuf, acc_sh.at[pl.ds(m * K, K)])

    zero_acc()
    plsc.subcore_barrier()

    for p, dst_v in enumerate((dst0_v, dst1_v, dst2_v)):
        pltpu.async_copy(h_hbm.at[src_v.at[0]], rows_a, sem_a)

        def pair(j2, carry, dst_v=dst_v):
            j = j2 * 2
            pltpu.make_async_copy(
                h_hbm.at[src_v.at[j]], rows_a, sem_a).wait()
            pltpu.async_copy(h_hbm.at[src_v.at[j + 1]], rows_b, sem_b)
            pltpu.sync_copy(rows_a, acc_sh.at[dst_v.at[j]], add=True)
            pltpu.make_async_copy(
                h_hbm.at[src_v.at[j + 1]], rows_b, sem_b).wait()
            pltpu.async_copy(h_hbm.at[src_v.at[j + 2]], rows_a, sem_a)
            pltpu.sync_copy(rows_b, acc_sh.at[dst_v.at[j + 1]], add=True)
            return carry
        lax.fori_loop(0, CH // 2, pair, None)
        pltpu.make_async_copy(
            h_hbm.at[src_v.at[CH - 1]], rows_a, sem_a).wait()
        pltpu.sync_copy(rows_a, acc_sh.at[dst_v.at[CH - 1]], add=True)

        plsc.subcore_barrier()

        rp = min(R, N - p * R)
        cp = rp // K
        for i in range(-(-cp // NS)):
            m = i * NS + s

            @pl.when(m < cp)
            def _(m=m):
                pltpu.sync_copy(acc_sh.at[pl.ds(m * K, K)], zbuf)
                pltpu.sync_copy(zbuf, out_hbm.at[c, pl.ds(p * R + m * K, K)])

        if p < NPASS - 1:
            lax.fori_loop(0, K, zrow, None)
            zero_acc()
            plsc.subcore_barrier()


_BLK = 2000
_GRID = N // _BLK


def _mm_scale_body(x_ref, w_ref, b_ref, degs_ref, o_ref):
    h = jnp.dot(x_ref[...], w_ref[...], preferred_element_type=f32) + b_ref[...]
    o_ref[...] = h * lax.rsqrt(jnp.maximum(degs_ref[...], 1.0))


def _tc_mm_scale(x, w, b2d, degs):
    return pl.pallas_call(
        _mm_scale_body,
        grid=(_GRID,),
        in_specs=[
            pl.BlockSpec((_BLK, D), lambda i: (i, 0)),
            pl.BlockSpec((D, D), lambda i: (0, 0)),
            pl.BlockSpec((1, D), lambda i: (0, 0)),
            pl.BlockSpec((_BLK, 1), lambda i: (i, 0)),
        ],
        out_specs=pl.BlockSpec((_BLK, D), lambda i: (i, 0)),
        out_shape=jax.ShapeDtypeStruct((N, D), f32),
    )(x, w, b2d, degs)


def _post_body(p_ref, degd_ref, o_ref):
    a = (p_ref[0] + p_ref[1]) * lax.rsqrt(jnp.maximum(degd_ref[...], 1.0))
    o_ref[...] = jnp.where(a >= 0, a, 0.01 * a)


def _tc_post(p, degd):
    return pl.pallas_call(
        _post_body,
        grid=(_GRID,),
        in_specs=[
            pl.BlockSpec((NC, _BLK, D), lambda i: (0, i, 0)),
            pl.BlockSpec((_BLK, 1), lambda i: (i, 0)),
        ],
        out_specs=pl.BlockSpec((_BLK, D), lambda i: (i, 0)),
        out_shape=jax.ShapeDtypeStruct((N, D), f32),
    )(p, degd)


def _fin_body(x0_ref, ys_ref, o_ref):
    o_ref[...] = (x0_ref[...] + ys_ref[0] + ys_ref[1]) * (1.0 / 3.0)


def _tc_fin(x0, ys):
    return pl.pallas_call(
        _fin_body,
        grid=(_GRID,),
        in_specs=[
            pl.BlockSpec((_BLK, D), lambda i: (i, 0)),
            pl.BlockSpec((2, _BLK, D), lambda i: (0, i, 0)),
        ],
        out_specs=pl.BlockSpec((_BLK, D), lambda i: (i, 0)),
        out_shape=jax.ShapeDtypeStruct((N, D), f32),
    )(x0, ys)


def kernel(edge_index, all_embed, W1, b1, W2, b2):
    ei = edge_index.astype(jnp.int32)
    deg_idx = ei.reshape(NW, EPT2)
    src_r = ei[0].reshape(NW, CH, K)
    dst_r = ei[1].reshape(NW, CH, K)

    _, degs = _deg_kernel(deg_idx)
    deg_src = degs[0, :N].reshape(N, 1)
    deg_dst = degs[1, :N].reshape(N, 1)
    Ws = jnp.stack((W1, W2))
    bs = jnp.stack((b1.reshape(1, D), b2.reshape(1, D)))

    def layer(x, wb):
        w, b2d = wb
        h = _tc_mm_scale(x, w, b2d, deg_src)
        p = _scatter_kernel(src_r, dst_r, h)
        xn = _tc_post(p, deg_dst)
        return xn, xn

    _, ys = lax.scan(layer, all_embed, (Ws, bs))
    return _tc_fin(all_embed, ys)


# speedup vs baseline: 5.2957x; 1.0981x over previous
import functools

import jax
import jax.numpy as jnp
from jax import lax
from jax.experimental import pallas as pl
from jax.experimental.pallas import tpu as pltpu
from jax.experimental.pallas import tpu_sc as plsc

N = 10000
E = 320000
D = 128
NC = 2
NS = 16
NW = NC * NS
NP = 10240
RPT_DEG = NP // NS
EPT2 = E // NS
K = 80
EPT = E // NW
CH = EPT // K
R = 5040
NPASS = 2
ACC = 5120
DUMP = R

f32 = jnp.float32

_mesh = plsc.VectorSubcoreMesh(
    core_axis_name="c", subcore_axis_name="s", num_cores=NC, num_subcores=NS)
_sc_params = pltpu.CompilerParams(needs_layout_passes=False)


@functools.partial(
    pl.kernel,
    out_type=[
        jax.ShapeDtypeStruct((NW, NP), f32),
        jax.ShapeDtypeStruct((2, NP), f32),
    ],
    mesh=_mesh,
    scratch_types=[
        pltpu.VMEM((EPT2,), jnp.int32),
        pltpu.VMEM((NP,), f32),
        pltpu.VMEM((RPT_DEG,), f32),
        pltpu.VMEM((RPT_DEG,), f32),
    ],
    compiler_params=_sc_params,
)
def _deg_kernel(idx_hbm, part_out, deg_out, idx_v, deg_v, acc_v, tmp_v):
    c = lax.axis_index("c")
    s = lax.axis_index("s")
    row = c * NS + s
    pltpu.sync_copy(idx_hbm.at[row], idx_v)

    zero16 = jnp.zeros((16,), f32)
    ones16 = jnp.ones((16,), f32)

    def zbody(i, carry):
        deg_v[pl.ds(i * 16, 16)] = zero16
        return carry
    lax.fori_loop(0, NP // 16, zbody, None)

    def ebody(e, carry):
        idx = idx_v[pl.ds(e * 16, 16)]
        plsc.addupdate_scatter(deg_v, [idx], ones16)
        return carry
    lax.fori_loop(0, EPT2 // 16, ebody, None)

    pltpu.sync_copy(deg_v, part_out.at[row])
    plsc.subcore_barrier()

    base = s * RPT_DEG
    pltpu.sync_copy(part_out.at[c * NS, pl.ds(base, RPT_DEG)], acc_v)
    for p in range(1, NS):
        pltpu.sync_copy(part_out.at[c * NS + p, pl.ds(base, RPT_DEG)], tmp_v)

        def abody(i, carry):
            sl = pl.ds(i * 16, 16)
            acc_v[sl] = acc_v[sl] + tmp_v[sl]
            return carry
        lax.fori_loop(0, RPT_DEG // 16, abody, None)
    pltpu.sync_copy(acc_v, deg_out.at[c, pl.ds(base, RPT_DEG)])


@functools.partial(
    pl.kernel,
    out_type=jax.ShapeDtypeStruct((NC, N, D), f32),
    mesh=_mesh,
    scratch_types=[
        pltpu.VMEM((CH, K), jnp.int32),
        pltpu.VMEM((CH, K), jnp.int32),
        pltpu.VMEM((CH, K), jnp.int32),
        pltpu.VMEM((K, D), f32),
        pltpu.VMEM((K, D), f32),
        pltpu.VMEM((K, D), f32),
        pltpu.VMEM_SHARED((ACC, D), f32),
        pltpu.SemaphoreType.DMA,
        pltpu.SemaphoreType.DMA,
    ],
    compiler_params=_sc_params,
)
def _scatter_kernel(src_hbm, dst_hbm, h_hbm, out_hbm,
                    src_v, dst0_v, dst1_v, rows_a, rows_b, zbuf,
                    acc_sh, sem_a, sem_b):
    c = lax.axis_index("c")
    s = lax.axis_index("s")
    w = c * NS + s
    pltpu.sync_copy(src_hbm.at[w], src_v)
    pltpu.sync_copy(dst_hbm.at[w], dst0_v)

    dumpv = jnp.full((16,), DUMP, jnp.int32)
    m63 = jnp.full((16,), 63, jnp.int32)
    r1 = jnp.full((16,), R, jnp.int32)

    def tbody(j, carry):
        for k in range(K // 16):
            sl = pl.ds(k * 16, 16)
            d = dst0_v[j, sl]
            dmp = dumpv + (d & m63)
            dst1_v[j, sl] = jnp.where(d >= r1, d - r1, dmp)
            dst0_v[j, sl] = jnp.where(d < r1, d, dmp)
        return carry
    lax.fori_loop(0, CH, tbody, None)

    zero16 = jnp.zeros((16,), f32)

    def zrow(i, carry):
        for j in range(D // 16):
            zbuf[i, pl.ds(j * 16, 16)] = zero16
        return carry
    lax.fori_loop(0, K, zrow, None)

    def zero_acc():
        for i in range(-(-(ACC // K) // NS)):
            m = i * NS + s

            @pl.when(m < ACC // K)
            def _():
                pltpu.sync_copy(zbuf, acc_sh.at[pl.ds(m * K, K)])

    zero_acc()
    plsc.subcore_barrier()

    for p, dst_v in enumerate((dst0_v, dst1_v)):
        pltpu.async_copy(h_hbm.at[src_v.at[0]], rows_a, sem_a)

        def pair(j2, carry, dst_v=dst_v):
            j = j2 * 2
            pltpu.make_async_copy(
                h_hbm.at[src_v.at[j]], rows_a, sem_a).wait()
            pltpu.async_copy(h_hbm.at[src_v.at[j + 1]], rows_b, sem_b)
            pltpu.sync_copy(rows_a, acc_sh.at[dst_v.at[j]], add=True)
            pltpu.make_async_copy(
                h_hbm.at[src_v.at[j + 1]], rows_b, sem_b).wait()
            pltpu.async_copy(h_hbm.at[src_v.at[j + 2]], rows_a, sem_a)
            pltpu.sync_copy(rows_b, acc_sh.at[dst_v.at[j + 1]], add=True)
            return carry
        lax.fori_loop(0, CH // 2, pair, None)
        pltpu.make_async_copy(
            h_hbm.at[src_v.at[CH - 1]], rows_a, sem_a).wait()
        pltpu.sync_copy(rows_a, acc_sh.at[dst_v.at[CH - 1]], add=True)

        plsc.subcore_barrier()

        rp = min(R, N - p * R)
        cp = rp // K
        for i in range(-(-cp // NS)):
            m = i * NS + s

            @pl.when(m < cp)
            def _(m=m):
                pltpu.sync_copy(acc_sh.at[pl.ds(m * K, K)], zbuf)
                pltpu.sync_copy(zbuf, out_hbm.at[c, pl.ds(p * R + m * K, K)])

        if p < NPASS - 1:
            lax.fori_loop(0, K, zrow, None)
            zero_acc()
            plsc.subcore_barrier()


_BLK = 2000
_GRID = N // _BLK


def _mm_scale_body(x_ref, w_ref, b_ref, degs_ref, o_ref):
    h = jnp.dot(x_ref[...], w_ref[...], preferred_element_type=f32) + b_ref[...]
    o_ref[...] = h * lax.rsqrt(jnp.maximum(degs_ref[...], 1.0))


def _tc_mm_scale(x, w, b2d, degs):
    return pl.pallas_call(
        _mm_scale_body,
        grid=(_GRID,),
        in_specs=[
            pl.BlockSpec((_BLK, D), lambda i: (i, 0)),
            pl.BlockSpec((D, D), lambda i: (0, 0)),
            pl.BlockSpec((1, D), lambda i: (0, 0)),
            pl.BlockSpec((_BLK, 1), lambda i: (i, 0)),
        ],
        out_specs=pl.BlockSpec((_BLK, D), lambda i: (i, 0)),
        out_shape=jax.ShapeDtypeStruct((N, D), f32),
    )(x, w, b2d, degs)


def _post_body(p_ref, degd_ref, o_ref):
    a = (p_ref[0] + p_ref[1]) * lax.rsqrt(jnp.maximum(degd_ref[...], 1.0))
    o_ref[...] = jnp.where(a >= 0, a, 0.01 * a)


def _tc_post(p, degd):
    return pl.pallas_call(
        _post_body,
        grid=(_GRID,),
        in_specs=[
            pl.BlockSpec((NC, _BLK, D), lambda i: (0, i, 0)),
            pl.BlockSpec((_BLK, 1), lambda i: (i, 0)),
        ],
        out_specs=pl.BlockSpec((_BLK, D), lambda i: (i, 0)),
        out_shape=jax.ShapeDtypeStruct((N, D), f32),
    )(p, degd)


def _fin_body(x0_ref, ys_ref, o_ref):
    o_ref[...] = (x0_ref[...] + ys_ref[0] + ys_ref[1]) * (1.0 / 3.0)


def _tc_fin(x0, ys):
    return pl.pallas_call(
        _fin_body,
        grid=(_GRID,),
        in_specs=[
            pl.BlockSpec((_BLK, D), lambda i: (i, 0)),
            pl.BlockSpec((2, _BLK, D), lambda i: (0, i, 0)),
        ],
        out_specs=pl.BlockSpec((_BLK, D), lambda i: (i, 0)),
        out_shape=jax.ShapeDtypeStruct((N, D), f32),
    )(x0, ys)


def kernel(edge_index, all_embed, W1, b1, W2, b2):
    ei = edge_index.astype(jnp.int32)
    deg_idx = ei.reshape(NW, EPT2)
    src_r = ei[0].reshape(NW, CH, K)
    dst_r = ei[1].reshape(NW, CH, K)

    _, degs = _deg_kernel(deg_idx)
    deg_src = degs[0, :N].reshape(N, 1)
    deg_dst = degs[1, :N].reshape(N, 1)
    Ws = jnp.stack((W1, W2))
    bs = jnp.stack((b1.reshape(1, D), b2.reshape(1, D)))

    def layer(x, wb):
        w, b2d = wb
        h = _tc_mm_scale(x, w, b2d, deg_src)
        p = _scatter_kernel(src_r, dst_r, h)
        xn = _tc_post(p, deg_dst)
        return xn, xn

    _, ys = lax.scan(layer, all_embed, (Ws, bs))
    return _tc_fin(all_embed, ys)


# speedup vs baseline: 5.4052x; 1.0207x over previous
import functools

import jax
import jax.numpy as jnp
from jax import lax
from jax.experimental import pallas as pl
from jax.experimental.pallas import tpu as pltpu
from jax.experimental.pallas import tpu_sc as plsc

N = 10000
E = 320000
D = 128
NC = 2
NS = 16
NW = NC * NS
NP = 10240
RPT_DEG = NP // NS
EPT2 = E // NS
K = 80
EPT = E // NW
CH = EPT // K
R = 5040
NPASS = 2
ACC = 5120
DUMP = R

f32 = jnp.float32

_mesh = plsc.VectorSubcoreMesh(
    core_axis_name="c", subcore_axis_name="s", num_cores=NC, num_subcores=NS)
_sc_params = pltpu.CompilerParams(needs_layout_passes=False)


@functools.partial(
    pl.kernel,
    out_type=[
        jax.ShapeDtypeStruct((NW, NP), f32),
        jax.ShapeDtypeStruct((2, NP), f32),
    ],
    mesh=_mesh,
    scratch_types=[
        pltpu.VMEM((EPT2,), jnp.int32),
        pltpu.VMEM((NP,), f32),
        pltpu.VMEM((RPT_DEG,), f32),
        pltpu.VMEM((RPT_DEG,), f32),
    ],
    compiler_params=_sc_params,
)
def _deg_kernel(idx_hbm, part_out, deg_out, idx_v, deg_v, acc_v, tmp_v):
    c = lax.axis_index("c")
    s = lax.axis_index("s")
    row = c * NS + s
    pltpu.sync_copy(idx_hbm.at[row], idx_v)

    zero16 = jnp.zeros((16,), f32)
    ones16 = jnp.ones((16,), f32)

    def zbody(i, carry):
        deg_v[pl.ds(i * 16, 16)] = zero16
        return carry
    lax.fori_loop(0, NP // 16, zbody, None)

    def ebody(e, carry):
        idx = idx_v[pl.ds(e * 16, 16)]
        plsc.addupdate_scatter(deg_v, [idx], ones16)
        return carry
    lax.fori_loop(0, EPT2 // 16, ebody, None)

    pltpu.sync_copy(deg_v, part_out.at[row])
    plsc.subcore_barrier()

    base = s * RPT_DEG
    pltpu.sync_copy(part_out.at[c * NS, pl.ds(base, RPT_DEG)], acc_v)
    for p in range(1, NS):
        pltpu.sync_copy(part_out.at[c * NS + p, pl.ds(base, RPT_DEG)], tmp_v)

        def abody(i, carry):
            sl = pl.ds(i * 16, 16)
            acc_v[sl] = acc_v[sl] + tmp_v[sl]
            return carry
        lax.fori_loop(0, RPT_DEG // 16, abody, None)
    pltpu.sync_copy(acc_v, deg_out.at[c, pl.ds(base, RPT_DEG)])


@functools.partial(
    pl.kernel,
    out_type=jax.ShapeDtypeStruct((NC, N, D), f32),
    mesh=_mesh,
    scratch_types=[
        pltpu.VMEM((CH, K), jnp.int32),
        pltpu.VMEM((CH, K), jnp.int32),
        pltpu.VMEM((CH, K), jnp.int32),
        pltpu.VMEM((K, D), f32),
        pltpu.VMEM((K, D), f32),
        pltpu.VMEM((K, D), f32),
        pltpu.VMEM_SHARED((ACC, D), f32),
        pltpu.SemaphoreType.DMA,
        pltpu.SemaphoreType.DMA,
    ],
    compiler_params=_sc_params,
)
def _scatter_kernel(src_hbm, dst_hbm, h_hbm, out_hbm,
                    src_v, dst0_v, dst1_v, rows_a, rows_b, zbuf,
                    acc_sh, sem_a, sem_b):
    c = lax.axis_index("c")
    s = lax.axis_index("s")
    w = c * NS + s
    pltpu.sync_copy(src_hbm.at[w], src_v)
    pltpu.sync_copy(dst_hbm.at[w], dst0_v)

    dumpv = jnp.full((16,), DUMP, jnp.int32)
    m63 = jnp.full((16,), 63, jnp.int32)
    r1 = jnp.full((16,), R, jnp.int32)

    def tbody(j, carry):
        for k in range(K // 16):
            sl = pl.ds(k * 16, 16)
            d = dst0_v[j, sl]
            dmp = dumpv + (d & m63)
            dst1_v[j, sl] = jnp.where(d >= r1, d - r1, dmp)
            dst0_v[j, sl] = jnp.where(d < r1, d, dmp)
        return carry
    lax.fori_loop(0, CH, tbody, None)

    zero16 = jnp.zeros((16,), f32)

    def zrow(i, carry):
        for j in range(D // 16):
            zbuf[i, pl.ds(j * 16, 16)] = zero16
        return carry
    lax.fori_loop(0, K, zrow, None)

    def zero_acc():
        for i in range(-(-(ACC // K) // NS)):
            m = i * NS + s

            @pl.when(m < ACC // K)
            def _():
                pltpu.sync_copy(zbuf, acc_sh.at[pl.ds(m * K, K)])

    zero_acc()
    plsc.subcore_barrier()

    for p, dst_v in enumerate((dst0_v, dst1_v)):
        pltpu.async_copy(h_hbm.at[src_v.at[0]], rows_a, sem_a)

        def pair(j2, carry, dst_v=dst_v):
            j = j2 * 2
            pltpu.make_async_copy(
                h_hbm.at[src_v.at[j]], rows_a, sem_a).wait()
            pltpu.async_copy(h_hbm.at[src_v.at[j + 1]], rows_b, sem_b)
            pltpu.sync_copy(rows_a, acc_sh.at[dst_v.at[j]], add=True)
            pltpu.make_async_copy(
                h_hbm.at[src_v.at[j + 1]], rows_b, sem_b).wait()
            pltpu.async_copy(h_hbm.at[src_v.at[j + 2]], rows_a, sem_a)
            pltpu.sync_copy(rows_b, acc_sh.at[dst_v.at[j + 1]], add=True)
            return carry
        lax.fori_loop(0, CH // 2, pair, None)
        pltpu.make_async_copy(
            h_hbm.at[src_v.at[CH - 1]], rows_a, sem_a).wait()
        pltpu.sync_copy(rows_a, acc_sh.at[dst_v.at[CH - 1]], add=True)

        plsc.subcore_barrier()

        rp = min(R, N - p * R)
        cp = rp // K
        for i in range(-(-cp // NS)):
            m = i * NS + s

            @pl.when(m < cp)
            def _(m=m):
                pltpu.sync_copy(acc_sh.at[pl.ds(m * K, K)], zbuf)
                pltpu.sync_copy(zbuf, out_hbm.at[c, pl.ds(p * R + m * K, K)])

        if p < NPASS - 1:
            lax.fori_loop(0, K, zrow, None)
            zero_acc()
            plsc.subcore_barrier()


_BLK = 2000
_GRID = N // _BLK


def _mm_scale_body(x_ref, w_ref, b_ref, degs_ref, o_ref):
    h = jnp.dot(x_ref[...], w_ref[...], preferred_element_type=f32) + b_ref[...]
    o_ref[...] = h * lax.rsqrt(jnp.maximum(degs_ref[...], 1.0))


def _tc_mm_scale(x, w, b2d, degs):
    return pl.pallas_call(
        _mm_scale_body,
        grid=(_GRID,),
        in_specs=[
            pl.BlockSpec((_BLK, D), lambda i: (i, 0)),
            pl.BlockSpec((D, D), lambda i: (0, 0)),
            pl.BlockSpec((1, D), lambda i: (0, 0)),
            pl.BlockSpec((_BLK, 1), lambda i: (i, 0)),
        ],
        out_specs=pl.BlockSpec((_BLK, D), lambda i: (i, 0)),
        out_shape=jax.ShapeDtypeStruct((N, D), f32),
    )(x, w, b2d, degs)


def _mid_body(p_ref, degd_ref, w_ref, b_ref, degs_ref, x1_ref, h2_ref):
    a = (p_ref[0] + p_ref[1]) * lax.rsqrt(jnp.maximum(degd_ref[...], 1.0))
    x1 = jnp.where(a >= 0, a, 0.01 * a)
    x1_ref[...] = x1
    h2 = jnp.dot(x1, w_ref[...], preferred_element_type=f32) + b_ref[...]
    h2_ref[...] = h2 * lax.rsqrt(jnp.maximum(degs_ref[...], 1.0))


def _tc_mid(p, degd, w, b2d, degs):
    return pl.pallas_call(
        _mid_body,
        grid=(_GRID,),
        in_specs=[
            pl.BlockSpec((NC, _BLK, D), lambda i: (0, i, 0)),
            pl.BlockSpec((_BLK, 1), lambda i: (i, 0)),
            pl.BlockSpec((D, D), lambda i: (0, 0)),
            pl.BlockSpec((1, D), lambda i: (0, 0)),
            pl.BlockSpec((_BLK, 1), lambda i: (i, 0)),
        ],
        out_specs=[
            pl.BlockSpec((_BLK, D), lambda i: (i, 0)),
            pl.BlockSpec((_BLK, D), lambda i: (i, 0)),
        ],
        out_shape=[
            jax.ShapeDtypeStruct((N, D), f32),
            jax.ShapeDtypeStruct((N, D), f32),
        ],
    )(p, degd, w, b2d, degs)


def _fin_body(q_ref, degd_ref, x0_ref, x1_ref, o_ref):
    a = (q_ref[0] + q_ref[1]) * lax.rsqrt(jnp.maximum(degd_ref[...], 1.0))
    x2 = jnp.where(a >= 0, a, 0.01 * a)
    o_ref[...] = (x0_ref[...] + x1_ref[...] + x2) * (1.0 / 3.0)


def _tc_fin(q, degd, x0, x1):
    return pl.pallas_call(
        _fin_body,
        grid=(_GRID,),
        in_specs=[
            pl.BlockSpec((NC, _BLK, D), lambda i: (0, i, 0)),
            pl.BlockSpec((_BLK, 1), lambda i: (i, 0)),
            pl.BlockSpec((_BLK, D), lambda i: (i, 0)),
            pl.BlockSpec((_BLK, D), lambda i: (i, 0)),
        ],
        out_specs=pl.BlockSpec((_BLK, D), lambda i: (i, 0)),
        out_shape=jax.ShapeDtypeStruct((N, D), f32),
    )(q, degd, x0, x1)


def kernel(edge_index, all_embed, W1, b1, W2, b2):
    ei = edge_index.astype(jnp.int32)
    deg_idx = ei.reshape(NW, EPT2)
    src_r = ei[0].reshape(NW, CH, K)
    dst_r = ei[1].reshape(NW, CH, K)

    _, degs = _deg_kernel(deg_idx)
    deg_src = degs[0, :N].reshape(N, 1)
    deg_dst = degs[1, :N].reshape(N, 1)
    b1r = b1.reshape(1, D)
    b2r = b2.reshape(1, D)

    h1 = _tc_mm_scale(all_embed, W1, b1r, deg_src)
    p = _scatter_kernel(src_r, dst_r, h1)
    x1, h2 = _tc_mid(p, deg_dst, W2, b2r, deg_src)
    q = _scatter_kernel(src_r, dst_r, h2)
    return _tc_fin(q, deg_dst, all_embed, x1)
